# Initial kernel scaffold; baseline (speedup 1.0000x reference)
import jax
import jax.numpy as jnp
from jax.experimental import pallas as pl


def kernel(q, mu, edge_index, rbf, unit_vectors, cutoff_values, W1, b1, W2, b2, FW1, FB1, FW2, FB2):
    raise NotImplementedError("write your pallas kernel here")


# speedup vs baseline: 6.4156x; 6.4156x over previous
import functools

import jax
import jax.numpy as jnp
from jax import lax
from jax.experimental import pallas as pl
from jax.experimental.pallas import tpu as pltpu
from jax.experimental.pallas import tpu_sc as plsc


def _filter_body(rbf_ref, cut_ref, uv_ref, fw1_ref, fb1_ref, fw2_ref, fb2_ref,
                 fq_ref, fr_ref, fmu_ref, u0_ref, u1_ref, u2_ref):
    h = jnp.dot(rbf_ref[...], fw1_ref[...], preferred_element_type=jnp.float32)
    h = h + fb1_ref[...]
    h = h * jax.nn.sigmoid(h)
    f = jnp.dot(h, fw2_ref[...], preferred_element_type=jnp.float32)
    f = f + fb2_ref[...]
    f = f * cut_ref[...]
    H = fq_ref.shape[1]
    fq_ref[...] = f[:, :H]
    fr_ref[...] = f[:, H:2 * H]
    fmu_ref[...] = f[:, 2 * H:]
    B = uv_ref.shape[0]
    u0_ref[...] = jnp.broadcast_to(uv_ref[:, 0:1], (B, 16))
    u1_ref[...] = jnp.broadcast_to(uv_ref[:, 1:2], (B, 16))
    u2_ref[...] = jnp.broadcast_to(uv_ref[:, 2:3], (B, 16))


def _run_filters(rbf, cutoff, uv, FW1, FB1, FW2, FB2, BE):
    E, NRBF = rbf.shape
    H = FW1.shape[1]
    grid = E // BE
    out = pl.pallas_call(
        _filter_body,
        grid=(grid,),
        in_specs=[
            pl.BlockSpec((BE, NRBF), lambda j: (j, 0)),
            pl.BlockSpec((BE, 1), lambda j: (j, 0)),
            pl.BlockSpec((BE, 3), lambda j: (j, 0)),
            pl.BlockSpec((NRBF, H), lambda j: (0, 0)),
            pl.BlockSpec((1, H), lambda j: (0, 0)),
            pl.BlockSpec((H, 3 * H), lambda j: (0, 0)),
            pl.BlockSpec((1, 3 * H), lambda j: (0, 0)),
        ],
        out_specs=[pl.BlockSpec((BE, H), lambda j: (j, 0))] * 3
        + [pl.BlockSpec((BE, 16), lambda j: (j, 0))] * 3,
        out_shape=[jax.ShapeDtypeStruct((E, H), jnp.float32)] * 3
        + [jax.ShapeDtypeStruct((E, 16), jnp.float32)] * 3,
    )(rbf, cutoff.reshape(E, 1), uv, FW1, FB1.reshape(1, H), FW2,
      FB2.reshape(1, 3 * H))
    return out


def _node_body(q_ref, mu_ref, w1_ref, b1_ref, w2_ref, b2_ref,
               xq_ref, xr_ref, xmu_ref, m0_ref, m1_ref, m2_ref):
    h = jnp.dot(q_ref[...], w1_ref[...], preferred_element_type=jnp.float32)
    h = h + b1_ref[...]
    h = h * jax.nn.sigmoid(h)
    x = jnp.dot(h, w2_ref[...], preferred_element_type=jnp.float32)
    x = x + b2_ref[...]
    H = xq_ref.shape[1]
    xq_ref[...] = x[:, :H]
    xr_ref[...] = x[:, H:2 * H]
    xmu_ref[...] = x[:, 2 * H:]
    m0_ref[...] = mu_ref[:, 0, :]
    m1_ref[...] = mu_ref[:, 1, :]
    m2_ref[...] = mu_ref[:, 2, :]


def _run_node(q, mu, W1, b1, W2, b2, BN):
    N, H = q.shape
    grid = N // BN
    out = pl.pallas_call(
        _node_body,
        grid=(grid,),
        in_specs=[
            pl.BlockSpec((BN, H), lambda j: (j, 0)),
            pl.BlockSpec((BN, 3, H), lambda j: (j, 0, 0)),
            pl.BlockSpec((H, 3 * H), lambda j: (0, 0)),
            pl.BlockSpec((1, 3 * H), lambda j: (0, 0)),
            pl.BlockSpec((3 * H, 3 * H), lambda j: (0, 0)),
            pl.BlockSpec((1, 3 * H), lambda j: (0, 0)),
        ],
        out_specs=[pl.BlockSpec((BN, H), lambda j: (j, 0))] * 6,
        out_shape=[jax.ShapeDtypeStruct((N, H), jnp.float32)] * 6,
    )(q, mu, W1, b1.reshape(1, 3 * H), W2, b2.reshape(1, 3 * H))
    return out


def _make_sc_kernel(N, E, H):
    C = 64
    RT = E // C
    R2 = RT // 2
    rows_pt = (-(-N // 16) + 7) // 8 * 8
    NP = rows_pt * 16
    mesh = plsc.VectorSubcoreMesh(core_axis_name="c", subcore_axis_name="s",
                                  num_cores=2, num_subcores=16)

    def body(src2d, tgt2d, xq, xr, xmu, mu0, mu1, mu2, fq, fr, fmu,
             u0, u1, u2, parts,
             idx_s, idx_t, ba, bb, bc, bu, acc, sem):
        cid = lax.axis_index("c")
        sid = lax.axis_index("s")
        row0 = sid * rows_pt
        nrows = (R2 // 16) + jnp.where(sid < (R2 % 16), 1, 0)
        lane0 = jnp.where(
            lax.broadcasted_iota(jnp.int32, (16,), 0) == 0, 1.0, 0.0
        ).astype(jnp.float32)

        def zero_ba(i, _):
            z = jnp.zeros((16,), jnp.float32)
            for j in range(H // 16):
                ba[i, pl.ds(16 * j, 16)] = z
            return 0

        def zero_acc():
            nfull = rows_pt // C
            for k in range(nfull):
                pltpu.sync_copy(ba.at[pl.ds(0, C)],
                                acc.at[pl.ds(row0 + k * C, C)])
            rem = rows_pt - nfull * C
            if rem:
                pltpu.sync_copy(ba.at[pl.ds(0, rem)],
                                acc.at[pl.ds(row0 + nfull * C, rem)])

        def chunk_q(t, _):
            r = cid * R2 + sid + 16 * t
            pltpu.sync_copy(src2d.at[r], idx_s)
            pltpu.sync_copy(tgt2d.at[r], idx_t)
            pltpu.async_copy(xq.at[idx_s], ba, sem).wait()
            pltpu.sync_copy(fq.at[pl.ds(r * C, C)], bb)

            def edge(i, _):
                for j in range(H // 16):
                    s = pl.ds(16 * j, 16)
                    ba[i, s] = ba[i, s] * bb[i, s]
                return 0
            lax.fori_loop(0, C, edge, 0)
            pltpu.sync_copy(ba, acc.at[idx_t], add=True)
            return 0

        def make_chunk_d(mud, ud):
            def chunk_d(t, _):
                r = cid * R2 + sid + 16 * t
                pltpu.sync_copy(src2d.at[r], idx_s)
                pltpu.sync_copy(tgt2d.at[r], idx_t)
                pltpu.async_copy(xr.at[idx_s], ba, sem).wait()
                pltpu.sync_copy(fr.at[pl.ds(r * C, C)], bb)
                pltpu.sync_copy(ud.at[pl.ds(r * C, C)], bu)

                def edge1(i, _):
                    ue = bu[i, pl.ds(0, 16)]
                    for j in range(H // 16):
                        s = pl.ds(16 * j, 16)
                        ba[i, s] = ue * (ba[i, s] * bb[i, s])
                    return 0
                lax.fori_loop(0, C, edge1, 0)

                pltpu.async_copy(xmu.at[idx_s], bb, sem).wait()
                pltpu.sync_copy(fmu.at[pl.ds(r * C, C)], bc)

                def edge2(i, _):
                    for j in range(H // 16):
                        s = pl.ds(16 * j, 16)
                        bb[i, s] = bb[i, s] * bc[i, s]
                    return 0
                lax.fori_loop(0, C, edge2, 0)

                pltpu.async_copy(mud.at[idx_s], bc, sem).wait()

                def edge3(i, _):
                    for j in range(H // 16):
                        s = pl.ds(16 * j, 16)
                        bc[i, s] = ba[i, s] + bc[i, s] * bb[i, s]
                    return 0
                lax.fori_loop(0, C, edge3, 0)
                pltpu.sync_copy(bc, acc.at[idx_t], add=True)
                return 0
            return chunk_d

        def chunk_deg(t, _):
            r = cid * R2 + sid + 16 * t
            pltpu.sync_copy(tgt2d.at[r], idx_t)
            pltpu.sync_copy(ba, acc.at[idx_t], add=True)
            return 0

        for p in range(5):
            lax.fori_loop(0, C, zero_ba, 0)
            zero_acc()
            if p == 4:
                def preset(i, _):
                    ba[i, pl.ds(0, 16)] = lane0
                    return 0
                lax.fori_loop(0, C, preset, 0)
            plsc.subcore_barrier()
            if p == 0:
                lax.fori_loop(0, nrows, chunk_q, 0)
            elif p < 4:
                mud = (mu0, mu1, mu2)[p - 1]
                ud = (u0, u1, u2)[p - 1]
                lax.fori_loop(0, nrows, make_chunk_d(mud, ud), 0)
            else:
                lax.fori_loop(0, nrows, chunk_deg, 0)
            plsc.subcore_barrier()
            pltpu.sync_copy(acc.at[pl.ds(row0, rows_pt)],
                            parts.at[cid, p, pl.ds(row0, rows_pt)])
            plsc.subcore_barrier()

    kfn = pl.kernel(
        body,
        out_type=jax.ShapeDtypeStruct((2, 5, NP, H), jnp.float32),
        mesh=mesh,
        compiler_params=pltpu.CompilerParams(needs_layout_passes=False),
        scratch_types=[
            pltpu.VMEM((C,), jnp.int32),
            pltpu.VMEM((C,), jnp.int32),
            pltpu.VMEM((C, H), jnp.float32),
            pltpu.VMEM((C, H), jnp.float32),
            pltpu.VMEM((C, H), jnp.float32),
            pltpu.VMEM((C, 16), jnp.float32),
            pltpu.VMEM_SHARED((NP, H), jnp.float32),
            pltpu.SemaphoreType.DMA,
        ],
    )
    return kfn


def _final_body(q_ref, m0_ref, m1_ref, m2_ref, parts_ref,
                qo_ref, o0_ref, o1_ref, o2_ref):
    H = q_ref.shape[1]
    deg = parts_ref[0, 4, :, 0] + parts_ref[1, 4, :, 0]
    inv = 1.0 / jnp.maximum(deg, 1.0)
    sq = parts_ref[0, 0, :, :H] + parts_ref[1, 0, :, :H]
    qo_ref[...] = q_ref[...] + sq * inv[:, None]
    for d, (mref, oref) in enumerate(((m0_ref, o0_ref), (m1_ref, o1_ref),
                                      (m2_ref, o2_ref))):
        sv = parts_ref[0, 1 + d, :, :H] + parts_ref[1, 1 + d, :, :H]
        oref[...] = mref[...] + sv * inv[:, None]


def _run_final(q, m0, m1, m2, parts, BN):
    N, H = q.shape
    W = parts.shape[-1]
    grid = N // BN
    out = pl.pallas_call(
        _final_body,
        grid=(grid,),
        in_specs=[
            pl.BlockSpec((BN, H), lambda j: (j, 0)),
            pl.BlockSpec((BN, H), lambda j: (j, 0)),
            pl.BlockSpec((BN, H), lambda j: (j, 0)),
            pl.BlockSpec((BN, H), lambda j: (j, 0)),
            pl.BlockSpec((2, 5, BN, W), lambda j: (0, 0, j, 0)),
        ],
        out_specs=[pl.BlockSpec((BN, H), lambda j: (j, 0))] * 4,
        out_shape=[jax.ShapeDtypeStruct((N, H), jnp.float32)] * 4,
    )(q, m0, m1, m2, parts)
    return out


def kernel(q, mu, edge_index, rbf, unit_vectors, cutoff_values,
           W1, b1, W2, b2, FW1, FB1, FW2, FB2):
    N, H = q.shape
    E = edge_index.shape[1]
    assert E % 256 == 0 and N % 16 == 0 and H % 16 == 0

    fq, fr, fmu, u0, u1, u2 = _run_filters(rbf, cutoff_values, unit_vectors,
                                           FW1, FB1, FW2, FB2, BE=2000)
    xq, xr, xmu, m0, m1, m2 = _run_node(q, mu, W1, b1, W2, b2, BN=400)

    src2d = edge_index[1].reshape(E // 64, 64)
    tgt2d = edge_index[0].reshape(E // 64, 64)

    sc = _make_sc_kernel(N, E, H)
    parts = sc(src2d, tgt2d, xq, xr, xmu, m0, m1, m2, fq, fr, fmu,
               u0, u1, u2)

    qo, o0, o1, o2 = _run_final(q, m0, m1, m2, parts, BN=400)
    mu_out = jnp.stack([o0, o1, o2], axis=1)
    return (qo, mu_out)
